# baseline (speedup 1.0000x reference)
import functools

import jax
import jax.numpy as jnp
from jax import lax
from jax.experimental import pallas as pl
from jax.experimental.pallas import tpu as pltpu
from jax.experimental.pallas import tpu_sc as plsc

B = 16384
D = 32

_info = plsc.get_sparse_core_info()
NC = _info.num_cores
NS = _info.num_subcores
L = _info.num_lanes
NW = NC * NS
BPW = B // NW
CHUNK = 128
NCH = BPW // CHUNK


def _sc_dot(data2, user_factors, item_factors):
    mesh = plsc.VectorSubcoreMesh(core_axis_name="c", subcore_axis_name="s")

    @functools.partial(
        pl.kernel,
        mesh=mesh,
        compiler_params=pltpu.CompilerParams(
            needs_layout_passes=False, use_tc_tiling_on_sc=False),
        out_type=jax.ShapeDtypeStruct((B,), jnp.float32),
        scratch_types=[
            pltpu.VMEM((2 * BPW,), jnp.int32),
            pltpu.VMEM((BPW,), jnp.int32),
            pltpu.VMEM((BPW,), jnp.int32),
            pltpu.VMEM((BPW, D), jnp.float32),
            pltpu.VMEM((BPW, D), jnp.float32),
            pltpu.VMEM((BPW * (D // 2),), jnp.float32),
            pltpu.VMEM((BPW,), jnp.float32),
            pltpu.SemaphoreType.DMA,
        ],
    )
    def k(d2_hbm, ut_hbm, it_hbm, out_hbm,
          data_v, idx_u, idx_i, rows_u, rows_v, part, out_v, sem):
        wid = lax.axis_index("s") * NC + lax.axis_index("c")

        pltpu.sync_copy(d2_hbm.at[wid], data_v)

        def split_body(g, carry):
            k0 = pl.multiple_of(g * L, L)
            two_k = 2 * k0 + 2 * lax.iota(jnp.int32, L)
            idx_u[pl.ds(k0, L)] = plsc.load_gather(data_v, [two_k])
            idx_i[pl.ds(k0, L)] = plsc.load_gather(data_v, [two_k + 1])
            return carry

        lax.fori_loop(0, BPW // L, split_body, 0)

        copies = []
        for c in range(NCH):
            copies.append(pltpu.async_copy(
                ut_hbm.at[idx_u.at[pl.ds(c * CHUNK, CHUNK)]],
                rows_u.at[pl.ds(c * CHUNK, CHUNK)], sem))
            copies.append(pltpu.async_copy(
                it_hbm.at[idx_i.at[pl.ds(c * CHUNK, CHUNK)]],
                rows_v.at[pl.ds(c * CHUNK, CHUNK)], sem))
        for cp in copies:
            cp.wait()

        def mul_body(r, carry):
            u0 = rows_u[r, pl.ds(0, L)]
            u1 = rows_u[r, pl.ds(L, L)]
            v0 = rows_v[r, pl.ds(0, L)]
            v1 = rows_v[r, pl.ds(L, L)]
            part[pl.ds(pl.multiple_of(r * L, L), L)] = u0 * v0 + u1 * v1
            return carry

        lax.fori_loop(0, BPW, mul_body, 0)

        def red_body(g, carry):
            row0 = pl.multiple_of(g * L, L)
            flat0 = row0 * L + lax.iota(jnp.int32, L) * L
            acc = jnp.zeros((L,), jnp.float32)
            for j in range(L):
                acc = acc + plsc.load_gather(part, [flat0 + j])
            out_v[pl.ds(row0, L)] = acc
            return carry

        lax.fori_loop(0, BPW // L, red_body, 0)

        base = pl.multiple_of(wid * BPW, BPW)
        pltpu.sync_copy(out_v, out_hbm.at[pl.ds(base, BPW)])

    return k(data2, user_factors, item_factors)


def kernel(data, user_factors, item_factors):
    data2 = data.astype(jnp.int32).reshape(NW, 2 * BPW)
    return _sc_dot(data2, user_factors, item_factors)


# speedup vs baseline: 2.6177x; 2.6177x over previous
import functools

import jax
import jax.numpy as jnp
from jax import lax
from jax.experimental import pallas as pl
from jax.experimental.pallas import tpu as pltpu
from jax.experimental.pallas import tpu_sc as plsc

B = 16384
D = 32
LANES = 128

_info = plsc.get_sparse_core_info()
NC = _info.num_cores
NS = _info.num_subcores
L = _info.num_lanes
NW = NC * NS
BPW = B // NW


def _sc_dot(data2, ut, it):
    mesh = plsc.VectorSubcoreMesh(core_axis_name="c", subcore_axis_name="s")

    @functools.partial(
        pl.kernel,
        mesh=mesh,
        compiler_params=pltpu.CompilerParams(
            needs_layout_passes=False, use_tc_tiling_on_sc=True),
        out_type=jax.ShapeDtypeStruct((B,), jnp.float32),
        scratch_types=[
            pltpu.VMEM((2 * BPW,), jnp.int32),
            pltpu.VMEM((2, D, LANES), jnp.float32),
            pltpu.VMEM((2, D, LANES), jnp.float32),
            pltpu.VMEM((BPW * L,), jnp.float32),
            pltpu.VMEM((BPW,), jnp.float32),
            pltpu.SemaphoreType.DMA,
            pltpu.SemaphoreType.DMA,
        ],
    )
    def k(d2_hbm, ut_hbm, it_hbm, out_hbm,
          idx_vm, win_u, win_v, part, out_v, sem0, sem1):
        wid = lax.axis_index("s") * NC + lax.axis_index("c")

        pltpu.sync_copy(d2_hbm.at[wid], idx_vm)

        lanes = lax.iota(jnp.int32, L)

        def scal2(r):
            pos = jnp.minimum(2 * r, 2 * BPW - 2)
            base = pl.multiple_of((pos // L) * L, L)
            vec = idx_vm[pl.ds(base, L)]
            off = pos - base
            zero = jnp.zeros((L,), jnp.int32)
            iu = jnp.sum(jnp.where(lanes == jnp.full((L,), off), vec, zero))
            ii = jnp.sum(
                jnp.where(lanes == jnp.full((L,), off + 1), vec, zero))
            return iu, ii

        def fetch(slot, sem, iu, ii):
            cu = pl.multiple_of((iu // LANES) * LANES, LANES)
            ci = pl.multiple_of((ii // LANES) * LANES, LANES)
            pltpu.async_copy(
                ut_hbm.at[:, pl.ds(cu, LANES)], win_u.at[slot], sem)
            pltpu.async_copy(
                it_hbm.at[:, pl.ds(ci, LANES)], win_v.at[slot], sem)

        def drain(slot, sem):
            pltpu.make_async_copy(
                ut_hbm.at[:, pl.ds(0, LANES)], win_u.at[slot], sem).wait()
            pltpu.make_async_copy(
                it_hbm.at[:, pl.ds(0, LANES)], win_v.at[slot], sem).wait()

        rows_hi = lanes + L

        def compute(r, slot, iu, ii):
            lane_u = jnp.full((L,), iu % LANES, jnp.int32)
            lane_i = jnp.full((L,), ii % LANES, jnp.int32)
            u_lo = plsc.load_gather(win_u.at[slot], [lanes, lane_u])
            u_hi = plsc.load_gather(win_u.at[slot], [rows_hi, lane_u])
            v_lo = plsc.load_gather(win_v.at[slot], [lanes, lane_i])
            v_hi = plsc.load_gather(win_v.at[slot], [rows_hi, lane_i])
            part[pl.ds(pl.multiple_of(r * L, L), L)] = (
                u_lo * v_lo + u_hi * v_hi)

        iu0, ii0 = scal2(0)
        fetch(0, sem0, iu0, ii0)

        def body(p, carry):
            iu_a, ii_a = carry
            r0 = 2 * p
            iu_b, ii_b = scal2(r0 + 1)
            fetch(1, sem1, iu_b, ii_b)
            drain(0, sem0)
            compute(r0, 0, iu_a, ii_a)

            iu_c, ii_c = scal2(r0 + 2)

            @pl.when(p < BPW // 2 - 1)
            def _():
                fetch(0, sem0, iu_c, ii_c)

            drain(1, sem1)
            compute(r0 + 1, 1, iu_b, ii_b)
            return (iu_c, ii_c)

        lax.fori_loop(0, BPW // 2, body, (iu0, ii0))

        def red_body(g, carry):
            row0 = pl.multiple_of(g * L, L)
            flat0 = row0 * L + lax.iota(jnp.int32, L) * L
            acc = jnp.zeros((L,), jnp.float32)
            for j in range(L):
                acc = acc + plsc.load_gather(part, [flat0 + j])
            out_v[pl.ds(row0, L)] = acc
            return carry

        lax.fori_loop(0, BPW // L, red_body, 0)

        base = pl.multiple_of(wid * BPW, BPW)
        pltpu.sync_copy(out_v, out_hbm.at[pl.ds(base, BPW)])

    return k(data2, ut, it)


def kernel(data, user_factors, item_factors):
    data2 = data.astype(jnp.int32).reshape(NW, 2 * BPW)
    return _sc_dot(data2, user_factors.T, item_factors.T)


# speedup vs baseline: 4.1249x; 1.5758x over previous
import functools

import jax
import jax.numpy as jnp
from jax import lax
from jax.experimental import pallas as pl
from jax.experimental.pallas import tpu as pltpu
from jax.experimental.pallas import tpu_sc as plsc

B = 16384
D = 32
LANES = 128

_info = plsc.get_sparse_core_info()
NC = _info.num_cores
NS = _info.num_subcores
L = _info.num_lanes
NW = NC * NS
BPW = B // NW
NB = 8


def _sc_dot(data2, ut, it):
    mesh = plsc.VectorSubcoreMesh(core_axis_name="c", subcore_axis_name="s")

    @functools.partial(
        pl.kernel,
        mesh=mesh,
        compiler_params=pltpu.CompilerParams(
            needs_layout_passes=False, use_tc_tiling_on_sc=True),
        out_type=jax.ShapeDtypeStruct((B,), jnp.float32),
        scratch_types=[
            pltpu.VMEM((2 * BPW,), jnp.int32),
            pltpu.VMEM((NB, D, LANES), jnp.float32),
            pltpu.VMEM((NB, D, LANES), jnp.float32),
            pltpu.VMEM((BPW * L,), jnp.float32),
            pltpu.VMEM((BPW,), jnp.float32),
            [pltpu.SemaphoreType.DMA] * NB,
        ],
    )
    def k(d2_hbm, ut_hbm, it_hbm, out_hbm,
          idx_vm, win_u, win_v, part, out_v, sems):
        wid = lax.axis_index("s") * NC + lax.axis_index("c")

        pltpu.sync_copy(d2_hbm.at[wid], idx_vm)

        lanes = lax.iota(jnp.int32, L)

        def scal2(r):
            pos = jnp.minimum(2 * r, 2 * BPW - 2)
            base = pl.multiple_of((pos // L) * L, L)
            vec = idx_vm[pl.ds(base, L)]
            off = pos - base
            zero = jnp.zeros((L,), jnp.int32)
            iu = jnp.sum(jnp.where(lanes == jnp.full((L,), off), vec, zero))
            ii = jnp.sum(
                jnp.where(lanes == jnp.full((L,), off + 1), vec, zero))
            return iu, ii

        def fetch(slot, sem, iu, ii):
            cu = pl.multiple_of((iu // LANES) * LANES, LANES)
            ci = pl.multiple_of((ii // LANES) * LANES, LANES)
            pltpu.async_copy(
                ut_hbm.at[:, pl.ds(cu, LANES)], win_u.at[slot], sem)
            pltpu.async_copy(
                it_hbm.at[:, pl.ds(ci, LANES)], win_v.at[slot], sem)

        def drain(slot, sem):
            pltpu.make_async_copy(
                ut_hbm.at[:, pl.ds(0, LANES)], win_u.at[slot], sem).wait()
            pltpu.make_async_copy(
                it_hbm.at[:, pl.ds(0, LANES)], win_v.at[slot], sem).wait()

        rows_hi = lanes + L

        def compute(r, slot, iu, ii):
            lane_u = jnp.full((L,), iu % LANES, jnp.int32)
            lane_i = jnp.full((L,), ii % LANES, jnp.int32)
            u_lo = plsc.load_gather(win_u.at[slot], [lanes, lane_u])
            u_hi = plsc.load_gather(win_u.at[slot], [rows_hi, lane_u])
            v_lo = plsc.load_gather(win_v.at[slot], [lanes, lane_i])
            v_hi = plsc.load_gather(win_v.at[slot], [rows_hi, lane_i])
            part[pl.ds(pl.multiple_of(r * L, L), L)] = (
                u_lo * v_lo + u_hi * v_hi)

        for r in range(NB - 1):
            iu, ii = scal2(r)
            fetch(r, sems[r], iu, ii)

        def body(b, carry):
            for s in range(NB):
                r = NB * b + s
                nslot = (s + NB - 1) % NB
                iu_n, ii_n = scal2(r + NB - 1)

                @pl.when(r + NB - 1 < BPW)
                def _():
                    fetch(nslot, sems[nslot], iu_n, ii_n)

                drain(s, sems[s])
                iu, ii = scal2(r)
                compute(r, s, iu, ii)
            return carry

        lax.fori_loop(0, BPW // NB, body, 0)

        def red_body(g, carry):
            row0 = pl.multiple_of(g * L, L)
            flat0 = row0 * L + lax.iota(jnp.int32, L) * L
            acc = jnp.zeros((L,), jnp.float32)
            for j in range(L):
                acc = acc + plsc.load_gather(part, [flat0 + j])
            out_v[pl.ds(row0, L)] = acc
            return carry

        lax.fori_loop(0, BPW // L, red_body, 0)

        base = pl.multiple_of(wid * BPW, BPW)
        pltpu.sync_copy(out_v, out_hbm.at[pl.ds(base, BPW)])

    return k(data2, ut, it)


def kernel(data, user_factors, item_factors):
    data2 = data.astype(jnp.int32).reshape(NW, 2 * BPW)
    return _sc_dot(data2, user_factors.T, item_factors.T)
